# baseline (speedup 1.0000x reference)
import jax
import jax.numpy as jnp
from jax import lax
from jax.experimental import pallas as pl
from jax.experimental.pallas import tpu as pltpu, tpu_sc as plsc

N_NODES = 10000
N_EDGES = 320000
D_FEAT = 128

NUM_CORES = 2
NUM_SUBCORES = 16
NUM_WORKERS = NUM_CORES * NUM_SUBCORES
EDGES_PER_WORKER = N_EDGES // NUM_WORKERS
CHUNK = 16
NUM_CHUNKS = EDGES_PER_WORKER // CHUNK
NBUF = 5
GROUPS = NUM_CHUNKS // NBUF
LANES = 16
WARM_GROUPS = 1


def _sc_body(h_hbm, u_hbm, v_hbm, out_hbm, h_sp, idx_u, idx_v, rows_u, rows_v,
             out_v, gsem_u, gsem_v, osem, hsem):
  wid = lax.axis_index("s") * NUM_CORES + lax.axis_index("c")
  base_w = pl.multiple_of(wid * EDGES_PER_WORKER, 8)
  lane_iota = lax.iota(jnp.int32, LANES)

  @pl.when(lax.axis_index("s") == 0)
  def _stage_h():
    pltpu.async_copy(h_hbm, h_sp, hsem)

  pltpu.sync_copy(u_hbm.at[pl.ds(base_w, EDGES_PER_WORKER)], idx_u)
  pltpu.sync_copy(v_hbm.at[pl.ds(base_w, EDGES_PER_WORKER)], idx_v)

  def issue_gathers_hbm(ci, b):
    off = pl.multiple_of(ci * CHUNK, 8)
    pltpu.async_copy(h_hbm.at[idx_u.at[pl.ds(off, CHUNK)]], rows_u.at[b],
                     gsem_u.at[b])
    pltpu.async_copy(h_hbm.at[idx_v.at[pl.ds(off, CHUNK)]], rows_v.at[b],
                     gsem_v.at[b])

  def issue_gathers(ci, b):
    off = pl.multiple_of(ci * CHUNK, 8)
    pltpu.async_copy(h_sp.at[idx_u.at[pl.ds(off, CHUNK)]], rows_u.at[b],
                     gsem_u.at[b])
    pltpu.async_copy(h_sp.at[idx_v.at[pl.ds(off, CHUNK)]], rows_v.at[b],
                     gsem_v.at[b])

  def wait_gathers(ci, b):
    off = pl.multiple_of(ci * CHUNK, 8)
    pltpu.make_async_copy(h_sp.at[idx_u.at[pl.ds(off, CHUNK)]],
                          rows_u.at[b], gsem_u.at[b]).wait()
    pltpu.make_async_copy(h_sp.at[idx_v.at[pl.ds(off, CHUNK)]],
                          rows_v.at[b], gsem_v.at[b]).wait()

  def out_slice(ci):
    return out_hbm.at[pl.ds(pl.multiple_of(base_w + ci * CHUNK, 8), CHUNK)]

  lane15 = jnp.full((LANES,), LANES - 1, jnp.int32)

  def compute_chunk(b):
    def edge_body(k, score):
      prods = [
          rows_u[b, k, pl.ds(j * LANES, LANES)] *
          rows_v[b, k, pl.ds(j * LANES, LANES)]
          for j in range(D_FEAT // LANES)
      ]
      while len(prods) > 1:
        prods = [a + c for a, c in zip(prods[::2], prods[1::2])]
      tot = lax.gather(
          plsc.cumsum(prods[0]), lane15[:, None],
          lax.GatherDimensionNumbers(offset_dims=(), collapsed_slice_dims=(0,),
                                     start_index_map=(0,)),
          (1,), mode=lax.GatherScatterMode.PROMISE_IN_BOUNDS)
      return jnp.where(lane_iota == k, tot, score)

    score = lax.fori_loop(0, LANES, edge_body,
                          jnp.zeros((LANES,), jnp.float32), unroll=1)
    out_v[b, pl.ds(0, LANES)] = score

  for b in range(NBUF):
    issue_gathers_hbm(b, b)

  def make_group(issuer):
    def group(g, _):
      for b in range(NBUF):
        ci = g * NBUF + b
        wait_gathers(ci, b)

        @pl.when(g > 0)
        def _wait_prev_out():
          pltpu.make_async_copy(out_v.at[b], out_slice(ci - NBUF),
                                osem.at[b]).wait()

        compute_chunk(b)
        pltpu.async_copy(out_v.at[b], out_slice(ci), osem.at[b])

        @pl.when(g < GROUPS - 1)
        def _issue_next():
          issuer(ci + NBUF, b)

      return 0

    return group

  lax.fori_loop(0, WARM_GROUPS, make_group(issue_gathers_hbm), 0)

  @pl.when(lax.axis_index("s") == 0)
  def _wait_h_staged():
    pltpu.make_async_copy(h_hbm, h_sp, hsem).wait()

  plsc.subcore_barrier()
  lax.fori_loop(WARM_GROUPS, GROUPS, make_group(issue_gathers), 0)

  for b in range(NBUF):
    ci = (GROUPS - 1) * NBUF + b
    pltpu.make_async_copy(out_v.at[b], out_slice(ci), osem.at[b]).wait()


@jax.jit
def kernel(h, u, v):
  mesh = plsc.VectorSubcoreMesh(core_axis_name="c", subcore_axis_name="s")
  return pl.kernel(
      _sc_body,
      out_type=jax.ShapeDtypeStruct((N_EDGES,), jnp.float32),
      mesh=mesh,
      compiler_params=pltpu.CompilerParams(needs_layout_passes=False),
      scratch_types=[
          pltpu.VMEM_SHARED((N_NODES, D_FEAT), jnp.float32),
          pltpu.VMEM((EDGES_PER_WORKER,), jnp.int32),
          pltpu.VMEM((EDGES_PER_WORKER,), jnp.int32),
          pltpu.VMEM((NBUF, CHUNK, D_FEAT), jnp.float32),
          pltpu.VMEM((NBUF, CHUNK, D_FEAT), jnp.float32),
          pltpu.VMEM((NBUF, CHUNK), jnp.float32),
          pltpu.SemaphoreType.DMA((NBUF,)),
          pltpu.SemaphoreType.DMA((NBUF,)),
          pltpu.SemaphoreType.DMA((NBUF,)),
          pltpu.SemaphoreType.DMA,
      ],
  )(h, u, v)


# speedup vs baseline: 1.0012x; 1.0012x over previous
import jax
import jax.numpy as jnp
from jax import lax
from jax.experimental import pallas as pl
from jax.experimental.pallas import tpu as pltpu, tpu_sc as plsc

N_NODES = 10000
N_EDGES = 320000
D_FEAT = 128

NUM_CORES = 2
NUM_SUBCORES = 16
NUM_WORKERS = NUM_CORES * NUM_SUBCORES
EDGES_PER_WORKER = N_EDGES // NUM_WORKERS
CHUNK = 16
NUM_CHUNKS = EDGES_PER_WORKER // CHUNK
NBUF = 5
GROUPS = NUM_CHUNKS // NBUF
LANES = 16
WARM_GROUPS = 2


def _sc_body(h_hbm, u_hbm, v_hbm, out_hbm, h_sp, idx_u, idx_v, rows_u, rows_v,
             out_v, gsem_u, gsem_v, osem, hsem):
  wid = lax.axis_index("s") * NUM_CORES + lax.axis_index("c")
  base_w = pl.multiple_of(wid * EDGES_PER_WORKER, 8)
  lane_iota = lax.iota(jnp.int32, LANES)

  @pl.when(lax.axis_index("s") == 0)
  def _stage_h():
    pltpu.async_copy(h_hbm, h_sp, hsem)

  pltpu.sync_copy(u_hbm.at[pl.ds(base_w, EDGES_PER_WORKER)], idx_u)
  pltpu.sync_copy(v_hbm.at[pl.ds(base_w, EDGES_PER_WORKER)], idx_v)

  def issue_gathers_hbm(ci, b):
    off = pl.multiple_of(ci * CHUNK, 8)
    pltpu.async_copy(h_hbm.at[idx_u.at[pl.ds(off, CHUNK)]], rows_u.at[b],
                     gsem_u.at[b])
    pltpu.async_copy(h_hbm.at[idx_v.at[pl.ds(off, CHUNK)]], rows_v.at[b],
                     gsem_v.at[b])

  def issue_gathers(ci, b):
    off = pl.multiple_of(ci * CHUNK, 8)
    pltpu.async_copy(h_sp.at[idx_u.at[pl.ds(off, CHUNK)]], rows_u.at[b],
                     gsem_u.at[b])
    pltpu.async_copy(h_sp.at[idx_v.at[pl.ds(off, CHUNK)]], rows_v.at[b],
                     gsem_v.at[b])

  def wait_gathers(ci, b):
    off = pl.multiple_of(ci * CHUNK, 8)
    pltpu.make_async_copy(h_sp.at[idx_u.at[pl.ds(off, CHUNK)]],
                          rows_u.at[b], gsem_u.at[b]).wait()
    pltpu.make_async_copy(h_sp.at[idx_v.at[pl.ds(off, CHUNK)]],
                          rows_v.at[b], gsem_v.at[b]).wait()

  def out_slice(ci):
    return out_hbm.at[pl.ds(pl.multiple_of(base_w + ci * CHUNK, 8), CHUNK)]

  lane15 = jnp.full((LANES,), LANES - 1, jnp.int32)

  def compute_chunk(b):
    def edge_body(k, score):
      prods = [
          rows_u[b, k, pl.ds(j * LANES, LANES)] *
          rows_v[b, k, pl.ds(j * LANES, LANES)]
          for j in range(D_FEAT // LANES)
      ]
      while len(prods) > 1:
        prods = [a + c for a, c in zip(prods[::2], prods[1::2])]
      tot = lax.gather(
          plsc.cumsum(prods[0]), lane15[:, None],
          lax.GatherDimensionNumbers(offset_dims=(), collapsed_slice_dims=(0,),
                                     start_index_map=(0,)),
          (1,), mode=lax.GatherScatterMode.PROMISE_IN_BOUNDS)
      return jnp.where(lane_iota == k, tot, score)

    score = lax.fori_loop(0, LANES, edge_body,
                          jnp.zeros((LANES,), jnp.float32), unroll=1)
    out_v[b, pl.ds(0, LANES)] = score

  for b in range(NBUF):
    issue_gathers_hbm(b, b)

  def make_group(issuer):
    def group(g, _):
      for b in range(NBUF):
        ci = g * NBUF + b
        wait_gathers(ci, b)

        @pl.when(g > 0)
        def _wait_prev_out():
          pltpu.make_async_copy(out_v.at[b], out_slice(ci - NBUF),
                                osem.at[b]).wait()

        compute_chunk(b)
        pltpu.async_copy(out_v.at[b], out_slice(ci), osem.at[b])

        @pl.when(g < GROUPS - 1)
        def _issue_next():
          issuer(ci + NBUF, b)

      return 0

    return group

  lax.fori_loop(0, WARM_GROUPS, make_group(issue_gathers_hbm), 0)

  @pl.when(lax.axis_index("s") == 0)
  def _wait_h_staged():
    pltpu.make_async_copy(h_hbm, h_sp, hsem).wait()

  plsc.subcore_barrier()
  lax.fori_loop(WARM_GROUPS, GROUPS, make_group(issue_gathers), 0)

  for b in range(NBUF):
    ci = (GROUPS - 1) * NBUF + b
    pltpu.make_async_copy(out_v.at[b], out_slice(ci), osem.at[b]).wait()


@jax.jit
def kernel(h, u, v):
  mesh = plsc.VectorSubcoreMesh(core_axis_name="c", subcore_axis_name="s")
  return pl.kernel(
      _sc_body,
      out_type=jax.ShapeDtypeStruct((N_EDGES,), jnp.float32),
      mesh=mesh,
      compiler_params=pltpu.CompilerParams(needs_layout_passes=False),
      scratch_types=[
          pltpu.VMEM_SHARED((N_NODES, D_FEAT), jnp.float32),
          pltpu.VMEM((EDGES_PER_WORKER,), jnp.int32),
          pltpu.VMEM((EDGES_PER_WORKER,), jnp.int32),
          pltpu.VMEM((NBUF, CHUNK, D_FEAT), jnp.float32),
          pltpu.VMEM((NBUF, CHUNK, D_FEAT), jnp.float32),
          pltpu.VMEM((NBUF, CHUNK), jnp.float32),
          pltpu.SemaphoreType.DMA((NBUF,)),
          pltpu.SemaphoreType.DMA((NBUF,)),
          pltpu.SemaphoreType.DMA((NBUF,)),
          pltpu.SemaphoreType.DMA,
      ],
  )(h, u, v)
